# Initial kernel scaffold; baseline (speedup 1.0000x reference)
import jax
import jax.numpy as jnp
from jax.experimental import pallas as pl


def kernel(x, enc_W1, enc_b1, enc_g1, enc_be1, enc_W2, enc_b2, enc_g2, enc_be2, enc_W3, enc_b3, codebooks, dec_W1, dec_b1, dec_g1, dec_be1, dec_W2, dec_b2, dec_g2, dec_be2, dec_W3, dec_b3):
    raise NotImplementedError("write your pallas kernel here")


# speedup vs baseline: 1.1073x; 1.1073x over previous
import functools

import jax
import jax.numpy as jnp
from jax import lax
from jax.experimental import pallas as pl
from jax.experimental.pallas import tpu as pltpu
from jax.experimental.pallas import tpu_sc as plsc

M = 4
IN_DIM = 512
DIM = 256
K = 1024
B = 4096
EPS = 1e-5
BT = 512
NB = B // BT

_SC_INFO = plsc.get_sparse_core_info()
_NC = _SC_INFO.num_cores
_NS = _SC_INFO.num_subcores
_NW = _NC * _NS
_ROWS_PER_W = (M * B) // _NW
_CHUNK = 128
_NCHUNK = _ROWS_PER_W // _CHUNK


def _bn(h, g, b):
    return (h / jnp.sqrt(1.0 + EPS)) * g + b


def _enc_body(x_ref, eW1, eb1, eg1, ebe1, eW2, eb2, eg2, ebe2, eW3, eb3, cbs,
              res_ref, gidx_ref):
    x = x_ref[...]
    for m in range(M):
        h = lax.dot_general(x, eW1[m], (((1,), (1,)), ((), ())),
                            preferred_element_type=jnp.float32) + eb1[m:m + 1, :]
        h = jnp.maximum(_bn(h, eg1[m:m + 1, :], ebe1[m:m + 1, :]), 0.0)
        h = lax.dot_general(h, eW2[m], (((1,), (1,)), ((), ())),
                            preferred_element_type=jnp.float32) + eb2[m:m + 1, :]
        h = jnp.maximum(_bn(h, eg2[m:m + 1, :], ebe2[m:m + 1, :]), 0.0)
        ze = lax.dot_general(h, eW3[m], (((1,), (1,)), ((), ())),
                             preferred_element_type=jnp.float32) + eb3[m:m + 1, :]
        res_ref[m] = ze
        emb = cbs[m]
        a = jnp.sum(ze * ze, axis=1)[:, None]
        bb = jnp.sum(emb * emb, axis=1)[None, :]
        c = lax.dot_general(ze, emb, (((1,), (1,)), ((), ())),
                            preferred_element_type=jnp.float32)
        dist = (a + bb) - 2.0 * c
        minv = jnp.min(dist, axis=1, keepdims=True)
        iota = lax.broadcasted_iota(jnp.int32, (BT, K), 1)
        nn = jnp.min(jnp.where(dist == minv, iota, K), axis=1)
        gidx_ref[m] = nn + m * K


def _dec_body(ce_ref, dW1, db1, dg1, dbe1, dW2, db2, dg2, dbe2, dW3, db3,
              xhat_ref):
    zq = ce_ref[0] + ce_ref[1] + ce_ref[2] + ce_ref[3]
    d = lax.dot_general(zq, dW1[...], (((1,), (1,)), ((), ())),
                        preferred_element_type=jnp.float32) + db1[...]
    d = jnp.maximum(_bn(d, dg1[...], dbe1[...]), 0.0)
    d = lax.dot_general(d, dW2[...], (((1,), (1,)), ((), ())),
                        preferred_element_type=jnp.float32) + db2[...]
    d = jnp.maximum(_bn(d, dg2[...], dbe2[...]), 0.0)
    xhat_ref[...] = lax.dot_general(d, dW3[...], (((1,), (1,)), ((), ())),
                                    preferred_element_type=jnp.float32) + db3[...]


_sc_mesh = plsc.VectorSubcoreMesh(core_axis_name="c", subcore_axis_name="s")


@functools.partial(
    pl.kernel,
    mesh=_sc_mesh,
    out_type=jax.ShapeDtypeStruct((M * B, DIM), jnp.float32),
    scratch_types=[
        pltpu.VMEM((_CHUNK,), jnp.int32),
        pltpu.VMEM((_CHUNK, DIM), jnp.float32),
        pltpu.SemaphoreType.DMA,
    ],
)
def _sc_gather(table_hbm, idx_hbm, out_hbm, idx_v, rows_v, sem):
    wid = lax.axis_index("s") * _NC + lax.axis_index("c")
    base = wid * _ROWS_PER_W
    for j in range(_NCHUNK):
        off = base + j * _CHUNK
        pltpu.sync_copy(idx_hbm.at[pl.ds(off, _CHUNK)], idx_v)
        pltpu.async_copy(table_hbm.at[idx_v], rows_v, sem).wait()
        pltpu.sync_copy(rows_v, out_hbm.at[pl.ds(off, _CHUNK)])


def kernel(x, enc_W1, enc_b1, enc_g1, enc_be1, enc_W2, enc_b2, enc_g2, enc_be2,
           enc_W3, enc_b3, codebooks, dec_W1, dec_b1, dec_g1, dec_be1,
           dec_W2, dec_b2, dec_g2, dec_be2, dec_W3, dec_b3):
    full = lambda shape: pl.BlockSpec(shape, lambda i: (0,) * len(shape))
    res, gidx = pl.pallas_call(
        _enc_body,
        grid=(NB,),
        in_specs=[
            pl.BlockSpec((BT, IN_DIM), lambda i: (i, 0)),
            full((M, 128, IN_DIM)), full((M, 128)), full((M, 128)), full((M, 128)),
            full((M, 256, 128)), full((M, 256)), full((M, 256)), full((M, 256)),
            full((M, DIM, 256)), full((M, DIM)),
            full((M, K, DIM)),
        ],
        out_specs=[
            pl.BlockSpec((M, BT, DIM), lambda i: (0, i, 0)),
            pl.BlockSpec((M, BT), lambda i: (0, i)),
        ],
        out_shape=[
            jax.ShapeDtypeStruct((M, B, DIM), jnp.float32),
            jax.ShapeDtypeStruct((M, B), jnp.int32),
        ],
    )(x, enc_W1, enc_b1, enc_g1, enc_be1, enc_W2, enc_b2, enc_g2, enc_be2,
      enc_W3, enc_b3, codebooks)

    table = codebooks.reshape(M * K, DIM)
    ce_flat = _sc_gather(table, gidx.reshape(M * B))
    ce = ce_flat.reshape(M, B, DIM)

    x_hat = pl.pallas_call(
        _dec_body,
        grid=(NB,),
        in_specs=[
            pl.BlockSpec((M, BT, DIM), lambda i: (0, i, 0)),
            full((256, DIM)), full((256,)), full((256,)), full((256,)),
            full((128, 256)), full((128,)), full((128,)), full((128,)),
            full((IN_DIM, 128)), full((IN_DIM,)),
        ],
        out_specs=pl.BlockSpec((BT, IN_DIM), lambda i: (i, 0)),
        out_shape=jax.ShapeDtypeStruct((B, IN_DIM), jnp.float32),
    )(ce, dec_W1, dec_b1, dec_g1, dec_be1, dec_W2, dec_b2, dec_g2, dec_be2,
      dec_W3, dec_b3)
    return (x_hat, res, ce)
